# Initial kernel scaffold; baseline (speedup 1.0000x reference)
import jax
import jax.numpy as jnp
from jax.experimental import pallas as pl


def kernel(input_ids, embed_weight):
    raise NotImplementedError("write your pallas kernel here")


# speedup vs baseline: 1.4252x; 1.4252x over previous
import functools

import jax
import jax.numpy as jnp
from jax import lax
from jax.experimental import pallas as pl
from jax.experimental.pallas import tpu as pltpu
from jax.experimental.pallas import tpu_sc as plsc


@functools.lru_cache(maxsize=None)
def _build_gather(n_tokens: int, hidden: int):
    info = plsc.get_sparse_core_info()
    nc, ns = info.num_cores, info.num_subcores
    nw = nc * ns
    assert n_tokens % nw == 0
    rows_per_w = n_tokens // nw
    chunk = 32
    n_chunks = rows_per_w // chunk

    mesh = plsc.VectorSubcoreMesh(core_axis_name="c", subcore_axis_name="s")

    @functools.partial(
        pl.kernel,
        mesh=mesh,
        out_type=jax.ShapeDtypeStruct((n_tokens, hidden), jnp.float32),
        scratch_types=[
            pltpu.VMEM((rows_per_w,), jnp.int32),
            pltpu.VMEM((chunk, hidden), jnp.float32),
            pltpu.SemaphoreType.DMA,
        ],
    )
    def gather_k(table_hbm, idx_hbm, out_hbm, idx_v, rows_v, sem):
        wid = lax.axis_index("s") * nc + lax.axis_index("c")
        base = wid * rows_per_w
        pltpu.sync_copy(idx_hbm.at[pl.ds(base, rows_per_w)], idx_v)

        def body(i, carry):
            off = i * chunk
            pltpu.async_copy(
                table_hbm.at[idx_v.at[pl.ds(off, chunk)]], rows_v, sem
            ).wait()
            pltpu.sync_copy(rows_v, out_hbm.at[pl.ds(base + off, chunk)])
            return carry

        lax.fori_loop(0, n_chunks, body, 0)

    return gather_k


def kernel(input_ids, embed_weight):
    bsz, seq = input_ids.shape
    vocab, hidden = embed_weight.shape
    ids = input_ids.reshape(-1).astype(jnp.int32)
    flat = _build_gather(bsz * seq, hidden)(embed_weight, ids)
    text_embeds = flat.reshape(bsz, seq, hidden)
    position_ids = jnp.broadcast_to(jnp.arange(seq, dtype=jnp.int32), (bsz, seq))
    lb_loss = jnp.zeros((1,), dtype=text_embeds.dtype)
    return (text_embeds, position_ids, lb_loss)


# speedup vs baseline: 1.5703x; 1.1018x over previous
import functools

import jax
import jax.numpy as jnp
from jax import lax
from jax.experimental import pallas as pl
from jax.experimental.pallas import tpu as pltpu
from jax.experimental.pallas import tpu_sc as plsc


@functools.lru_cache(maxsize=None)
def _build_gather(n_tokens: int, hidden: int):
    info = plsc.get_sparse_core_info()
    nc, ns = info.num_cores, info.num_subcores
    nw = nc * ns
    assert n_tokens % nw == 0
    rows_per_w = n_tokens // nw
    chunk = 32
    n_chunks = rows_per_w // chunk

    mesh = plsc.VectorSubcoreMesh(core_axis_name="c", subcore_axis_name="s")

    @functools.partial(
        pl.kernel,
        mesh=mesh,
        out_type=jax.ShapeDtypeStruct((n_tokens, hidden), jnp.float32),
        scratch_types=[
            pltpu.VMEM((rows_per_w,), jnp.int32),
            pltpu.VMEM((chunk, hidden), jnp.float32),
            pltpu.VMEM((chunk, hidden), jnp.float32),
            pltpu.SemaphoreType.DMA,
            pltpu.SemaphoreType.DMA,
            pltpu.SemaphoreType.DMA,
            pltpu.SemaphoreType.DMA,
        ],
    )
    def gather_k(table_hbm, idx_hbm, out_hbm, idx_v, buf0, buf1, g0, g1, s0, s1):
        wid = lax.axis_index("s") * nc + lax.axis_index("c")
        base = wid * rows_per_w
        pltpu.sync_copy(idx_hbm.at[pl.ds(base, rows_per_w)], idx_v)

        def gather_start(i, buf, sem):
            pltpu.async_copy(table_hbm.at[idx_v.at[pl.ds(i * chunk, chunk)]], buf, sem)

        def gather_wait(i, buf, sem):
            pltpu.make_async_copy(
                table_hbm.at[idx_v.at[pl.ds(i * chunk, chunk)]], buf, sem
            ).wait()

        def scatter_start(i, buf, sem):
            pltpu.async_copy(buf, out_hbm.at[pl.ds(base + i * chunk, chunk)], sem)

        def scatter_wait(i, buf, sem):
            pltpu.make_async_copy(
                buf, out_hbm.at[pl.ds(base + i * chunk, chunk)], sem
            ).wait()

        n_groups = n_chunks // 2
        gather_start(0, buf0, g0)

        def body(t, carry):
            i0 = 2 * t
            i1 = i0 + 1
            gather_wait(i0, buf0, g0)
            scatter_start(i0, buf0, s0)

            @pl.when(t > 0)
            def _():
                scatter_wait(i1 - 2, buf1, s1)

            gather_start(i1, buf1, g1)
            gather_wait(i1, buf1, g1)
            scatter_start(i1, buf1, s1)

            @pl.when(t + 1 < n_groups)
            def _():
                scatter_wait(i0, buf0, s0)
                gather_start(i0 + 2, buf0, g0)

            return carry

        lax.fori_loop(0, n_groups, body, 0)

        scatter_wait(n_chunks - 2, buf0, s0)
        scatter_wait(n_chunks - 1, buf1, s1)

    return gather_k


def kernel(input_ids, embed_weight):
    bsz, seq = input_ids.shape
    vocab, hidden = embed_weight.shape
    ids = input_ids.reshape(-1).astype(jnp.int32)
    flat = _build_gather(bsz * seq, hidden)(embed_weight, ids)
    text_embeds = flat.reshape(bsz, seq, hidden)
    position_ids = jnp.broadcast_to(jnp.arange(seq, dtype=jnp.int32), (bsz, seq))
    lb_loss = jnp.zeros((1,), dtype=text_embeds.dtype)
    return (text_embeds, position_ids, lb_loss)
